# baseline (speedup 1.0000x reference)
import dataclasses
import functools

import jax
import jax.numpy as jnp
from jax import lax
from jax.experimental import pallas as pl
from jax.experimental.pallas import tpu as pltpu
from jax.experimental.pallas import tpu_sc as plsc

N = 10000
NP = 10240
E = 320000
D = 128
L = 16
NC = 2
NS = 16
NW = NC * NS
EPT = E // NW
CHUNK = 80
NCHUNK = EPT // CHUNK
GROUPS = CHUNK // L
RPT = NP // NS


def _matmul_body(x_ref, w_ref, o_ref):
    o_ref[...] = jnp.dot(x_ref[...], w_ref[...],
                         preferred_element_type=jnp.float32)


def _transform(x, W):
    return pl.pallas_call(
        _matmul_body,
        out_shape=jax.ShapeDtypeStruct((N, D), jnp.float32),
        grid=(10,),
        in_specs=[
            pl.BlockSpec((N // 10, D), lambda i: (i, 0)),
            pl.BlockSpec((D, D), lambda i: (0, 0)),
        ],
        out_specs=pl.BlockSpec((N // 10, D), lambda i: (i, 0)),
    )(x, W)


_SC_PARAMS = pltpu.CompilerParams()
if "needs_layout_passes" in pltpu.CompilerParams.__dataclass_fields__:
    _SC_PARAMS = dataclasses.replace(_SC_PARAMS, needs_layout_passes=False)


@functools.partial(
    pl.kernel,
    compiler_params=_SC_PARAMS,
    out_type=(
        jax.ShapeDtypeStruct((NC, NP, D), jnp.float32),
        jax.ShapeDtypeStruct((NC, NP, L), jnp.float32),
    ),
    mesh=plsc.VectorSubcoreMesh(core_axis_name="c", subcore_axis_name="s",
                                num_cores=NC, num_subcores=NS),
    scratch_types=[
        pltpu.VMEM_SHARED((NP, D), jnp.float32),
        pltpu.VMEM_SHARED((NP, L), jnp.float32),
        pltpu.VMEM((CHUNK,), jnp.int32),
        pltpu.VMEM((CHUNK,), jnp.int32),
        pltpu.VMEM((CHUNK,), jnp.int32),
        pltpu.VMEM((CHUNK, D), jnp.float32),
        pltpu.VMEM((CHUNK, D), jnp.float32),
        pltpu.VMEM((CHUNK, L), jnp.float32),
        pltpu.SemaphoreType.DMA,
        pltpu.SemaphoreType.DMA,
        pltpu.SemaphoreType.DMA,
    ],
)
def _edge_kernel(y_hbm, x_hbm, src_hbm, dst_hbm, agg_out, sum_out,
                 agg_sh, sum_sh, src_v, dst_v, win_v, yrows, xrows, estage,
                 isem, gsem, ssem):
    cid = lax.axis_index("c")
    sid = lax.axis_index("s")
    wid = sid * NC + cid

    zero16 = jnp.zeros((L,), jnp.float32)
    iota16 = lax.iota(jnp.int32, L)
    zero16i = jnp.zeros((L,), jnp.int32)
    row0 = sid * RPT

    @pl.loop(0, CHUNK)
    def _(i):
        for t in range(D // L):
            yrows[i, pl.ds(L * t, L)] = zero16
        estage[i, :] = zero16

    @pl.loop(0, RPT // CHUNK)
    def _(r):
        w0 = row0 + CHUNK * r
        for t in range(CHUNK // L):
            win_v[pl.ds(L * t, L)] = iota16 + (w0 + L * t)
        pltpu.sync_copy(yrows, agg_sh.at[win_v])
        pltpu.sync_copy(estage, sum_sh.at[win_v])
    plsc.subcore_barrier()

    @pl.loop(0, NCHUNK)
    def _(c):
        base = wid * EPT + c * CHUNK
        i1 = pltpu.make_async_copy(src_hbm.at[pl.ds(base, CHUNK)], src_v, isem)
        i2 = pltpu.make_async_copy(dst_hbm.at[pl.ds(base, CHUNK)], dst_v, isem)
        i1.start()
        i2.start()
        i1.wait()
        i2.wait()
        g1 = pltpu.make_async_copy(y_hbm.at[src_v], yrows, gsem)
        g2 = pltpu.make_async_copy(x_hbm.at[dst_v], xrows, gsem)
        g1.start()
        g2.start()
        g1.wait()
        g2.wait()
        for g in range(GROUPS):
            rowids = iota16 + (g * L)

            def dot_body(k, acc):
                colk = jnp.full((L,), 0, jnp.int32) + k
                a = plsc.load_gather(yrows, [rowids, colk])
                b = plsc.load_gather(xrows, [rowids, colk])
                return acc + a * b

            acc = lax.fori_loop(0, D, dot_body, zero16, unroll=8)
            e16 = jnp.exp(acc)
            plsc.store_scatter(estage, [rowids, zero16i], e16)

            def scale_body(k, e16):
                colk = jnp.full((L,), 0, jnp.int32) + k
                v = plsc.load_gather(yrows, [rowids, colk])
                plsc.store_scatter(yrows, [rowids, colk], v * e16)
                return e16

            lax.fori_loop(0, D, scale_body, e16, unroll=8)
        s1 = pltpu.make_async_copy(yrows, agg_sh.at[dst_v], ssem)
        s2 = pltpu.make_async_copy(estage, sum_sh.at[dst_v], ssem)
        s1.start(add=True)
        s2.start(add=True)
        s1.wait()
        s2.wait()

    plsc.subcore_barrier()

    @pl.loop(0, RPT // CHUNK)
    def _(r):
        w0 = row0 + CHUNK * r
        for t in range(CHUNK // L):
            win_v[pl.ds(L * t, L)] = iota16 + (w0 + L * t)
        pltpu.sync_copy(agg_sh.at[win_v], yrows)
        pltpu.sync_copy(sum_sh.at[win_v], estage)
        pltpu.sync_copy(yrows, agg_out.at[cid, pl.ds(w0, CHUNK)])
        pltpu.sync_copy(estage, sum_out.at[cid, pl.ds(w0, CHUNK)])


def _combine_body(pa_ref, ps_ref, o_ref):
    agg = pa_ref[0] + pa_ref[1]
    s = ps_ref[0, :, 0:1] + ps_ref[1, :, 0:1]
    agg = agg / (s + 1e-16)
    n2 = jnp.sum(agg * agg, axis=-1, keepdims=True)
    o_ref[...] = agg * (n2 / (1.0 + n2) / jnp.sqrt(n2 + 1e-9))


def _combine(pa, ps):
    return pl.pallas_call(
        _combine_body,
        out_shape=jax.ShapeDtypeStruct((NP, D), jnp.float32),
        grid=(5,),
        in_specs=[
            pl.BlockSpec((NC, NP // 5, D), lambda i: (0, i, 0)),
            pl.BlockSpec((NC, NP // 5, L), lambda i: (0, i, 0)),
        ],
        out_specs=pl.BlockSpec((NP // 5, D), lambda i: (i, 0)),
    )(pa, ps)


def kernel(x, edge_index, W):
    ei = edge_index.astype(jnp.int32)
    src = ei[0]
    dst = ei[1]
    y = _transform(x, W)
    pa, ps = _edge_kernel(y, x, src, dst)
    return _combine(pa, ps)[:N]


# speedup vs baseline: 1.8310x; 1.8310x over previous
import dataclasses
import functools

import jax
import jax.numpy as jnp
from jax import lax
from jax.experimental import pallas as pl
from jax.experimental.pallas import tpu as pltpu
from jax.experimental.pallas import tpu_sc as plsc

N = 10000
NP = 10240
E = 320000
D = 128
L = 16
NC = 2
NS = 16
NW = NC * NS
EPT = E // NW
CHUNK = 80
NCHUNK = EPT // CHUNK
GROUPS = CHUNK // L
RPT = NP // NS


def _matmul_body(x_ref, w_ref, o_ref):
    o_ref[...] = jnp.dot(x_ref[...], w_ref[...],
                         preferred_element_type=jnp.float32)


def _transform(x, W):
    return pl.pallas_call(
        _matmul_body,
        out_shape=jax.ShapeDtypeStruct((N, D), jnp.float32),
        grid=(10,),
        in_specs=[
            pl.BlockSpec((N // 10, D), lambda i: (i, 0)),
            pl.BlockSpec((D, D), lambda i: (0, 0)),
        ],
        out_specs=pl.BlockSpec((N // 10, D), lambda i: (i, 0)),
    )(x, W)


_SC_PARAMS = pltpu.CompilerParams()
if "needs_layout_passes" in pltpu.CompilerParams.__dataclass_fields__:
    _SC_PARAMS = dataclasses.replace(_SC_PARAMS, needs_layout_passes=False)


@functools.partial(
    pl.kernel,
    compiler_params=_SC_PARAMS,
    out_type=(
        jax.ShapeDtypeStruct((NC, NP, D), jnp.float32),
        jax.ShapeDtypeStruct((NC, NP, L), jnp.float32),
    ),
    mesh=plsc.VectorSubcoreMesh(core_axis_name="c", subcore_axis_name="s",
                                num_cores=NC, num_subcores=NS),
    scratch_types=[
        pltpu.VMEM_SHARED((NP, D), jnp.float32),
        pltpu.VMEM_SHARED((NP, L), jnp.float32),
        pltpu.VMEM((CHUNK,), jnp.int32),
        pltpu.VMEM((CHUNK,), jnp.int32),
        pltpu.VMEM((CHUNK,), jnp.int32),
        pltpu.VMEM((CHUNK, D), jnp.float32),
        pltpu.VMEM((CHUNK, D), jnp.float32),
        pltpu.VMEM((CHUNK, L), jnp.float32),
        pltpu.SemaphoreType.DMA,
        pltpu.SemaphoreType.DMA,
        pltpu.SemaphoreType.DMA,
    ],
)
def _edge_kernel(y_hbm, x_hbm, src_hbm, dst_hbm, agg_out, sum_out,
                 agg_sh, sum_sh, src_v, dst_v, win_v, yrows, xrows, estage,
                 isem, gsem, ssem):
    cid = lax.axis_index("c")
    sid = lax.axis_index("s")
    wid = sid * NC + cid

    zero16 = jnp.zeros((L,), jnp.float32)
    iota16 = lax.iota(jnp.int32, L)
    zero16i = jnp.zeros((L,), jnp.int32)
    row0 = sid * RPT

    @pl.loop(0, CHUNK)
    def _(i):
        for t in range(D // L):
            yrows[i, pl.ds(L * t, L)] = zero16
        estage[i, :] = zero16

    @pl.loop(0, RPT // CHUNK)
    def _(r):
        w0 = row0 + CHUNK * r
        for t in range(CHUNK // L):
            win_v[pl.ds(L * t, L)] = iota16 + (w0 + L * t)
        pltpu.sync_copy(yrows, agg_sh.at[win_v])
        pltpu.sync_copy(estage, sum_sh.at[win_v])
    plsc.subcore_barrier()

    @pl.loop(0, NCHUNK)
    def _(c):
        base = wid * EPT + c * CHUNK
        i1 = pltpu.make_async_copy(src_hbm.at[pl.ds(base, CHUNK)], src_v, isem)
        i2 = pltpu.make_async_copy(dst_hbm.at[pl.ds(base, CHUNK)], dst_v, isem)
        i1.start()
        i2.start()
        i1.wait()
        i2.wait()
        g1 = pltpu.make_async_copy(y_hbm.at[src_v], yrows, gsem)
        g2 = pltpu.make_async_copy(x_hbm.at[dst_v], xrows, gsem)
        g1.start()
        g2.start()
        g1.wait()
        g2.wait()
        for g in range(GROUPS):
            rowids = iota16 + (g * L)

            def dot_body(k, acc):
                colk = jnp.full((L,), 0, jnp.int32) + k
                a = plsc.load_gather(yrows, [rowids, colk])
                b = plsc.load_gather(xrows, [rowids, colk])
                return acc + a * b

            acc = lax.fori_loop(0, D, dot_body, zero16, unroll=8)
            e16 = jnp.exp(acc)
            plsc.store_scatter(estage, [rowids, zero16i], e16)
            for j in range(L):
                ej = g * L + j
                eb = plsc.load_gather(
                    estage, [jnp.full((L,), ej, jnp.int32), zero16i])
                for t in range(D // L):
                    sl = pl.ds(L * t, L)
                    yrows[ej, sl] = yrows[ej, sl] * eb
        s1 = pltpu.make_async_copy(yrows, agg_sh.at[dst_v], ssem)
        s2 = pltpu.make_async_copy(estage, sum_sh.at[dst_v], ssem)
        s1.start(add=True)
        s2.start(add=True)
        s1.wait()
        s2.wait()

    plsc.subcore_barrier()

    @pl.loop(0, RPT // CHUNK)
    def _(r):
        w0 = row0 + CHUNK * r
        for t in range(CHUNK // L):
            win_v[pl.ds(L * t, L)] = iota16 + (w0 + L * t)
        pltpu.sync_copy(agg_sh.at[win_v], yrows)
        pltpu.sync_copy(sum_sh.at[win_v], estage)
        pltpu.sync_copy(yrows, agg_out.at[cid, pl.ds(w0, CHUNK)])
        pltpu.sync_copy(estage, sum_out.at[cid, pl.ds(w0, CHUNK)])


def _combine_body(pa_ref, ps_ref, o_ref):
    agg = pa_ref[0] + pa_ref[1]
    s = ps_ref[0, :, 0:1] + ps_ref[1, :, 0:1]
    agg = agg / (s + 1e-16)
    n2 = jnp.sum(agg * agg, axis=-1, keepdims=True)
    o_ref[...] = agg * (n2 / (1.0 + n2) / jnp.sqrt(n2 + 1e-9))


def _combine(pa, ps):
    return pl.pallas_call(
        _combine_body,
        out_shape=jax.ShapeDtypeStruct((NP, D), jnp.float32),
        grid=(5,),
        in_specs=[
            pl.BlockSpec((NC, NP // 5, D), lambda i: (0, i, 0)),
            pl.BlockSpec((NC, NP // 5, L), lambda i: (0, i, 0)),
        ],
        out_specs=pl.BlockSpec((NP // 5, D), lambda i: (i, 0)),
    )(pa, ps)


def kernel(x, edge_index, W):
    ei = edge_index.astype(jnp.int32)
    src = ei[0]
    dst = ei[1]
    y = _transform(x, W)
    pa, ps = _edge_kernel(y, x, src, dst)
    return _combine(pa, ps)[:N]


# speedup vs baseline: 2.0093x; 1.0974x over previous
import dataclasses
import functools

import jax
import jax.numpy as jnp
from jax import lax
from jax.experimental import pallas as pl
from jax.experimental.pallas import tpu as pltpu
from jax.experimental.pallas import tpu_sc as plsc

N = 10000
NP = 10240
E = 320000
EP = E + 1024
D = 128
L = 16
NC = 2
NS = 16
NW = NC * NS
CHUNK = 32
NCHUNK = 312
NPAIR = NCHUNK // 2
GROUPS = CHUNK // L
RPT = NP // NS


def _matmul_body(x_ref, w_ref, o_ref):
    o_ref[...] = jnp.dot(x_ref[...], w_ref[...],
                         preferred_element_type=jnp.float32)


def _transform(x, W):
    return pl.pallas_call(
        _matmul_body,
        out_shape=jax.ShapeDtypeStruct((N, D), jnp.float32),
        grid=(10,),
        in_specs=[
            pl.BlockSpec((N // 10, D), lambda i: (i, 0)),
            pl.BlockSpec((D, D), lambda i: (0, 0)),
        ],
        out_specs=pl.BlockSpec((N // 10, D), lambda i: (i, 0)),
    )(x, W)


_SC_PARAMS = pltpu.CompilerParams()
if "needs_layout_passes" in pltpu.CompilerParams.__dataclass_fields__:
    _SC_PARAMS = dataclasses.replace(_SC_PARAMS, needs_layout_passes=False)


@functools.partial(
    pl.kernel,
    compiler_params=_SC_PARAMS,
    out_type=(
        jax.ShapeDtypeStruct((NC, NP, D), jnp.float32),
        jax.ShapeDtypeStruct((NC, NP, L), jnp.float32),
    ),
    mesh=plsc.VectorSubcoreMesh(core_axis_name="c", subcore_axis_name="s",
                                num_cores=NC, num_subcores=NS),
    scratch_types=[
        pltpu.VMEM_SHARED((NP, D), jnp.float32),
        pltpu.VMEM_SHARED((NP, L), jnp.float32),
        pltpu.VMEM((CHUNK,), jnp.int32),
        pltpu.VMEM((CHUNK,), jnp.int32),
        pltpu.VMEM((CHUNK,), jnp.int32),
        pltpu.VMEM((CHUNK,), jnp.int32),
        pltpu.VMEM((CHUNK,), jnp.int32),
        pltpu.VMEM((CHUNK,), jnp.int32),
        pltpu.VMEM((CHUNK,), jnp.int32),
        pltpu.VMEM((CHUNK, D), jnp.float32),
        pltpu.VMEM((CHUNK, D), jnp.float32),
        pltpu.VMEM((CHUNK, D), jnp.float32),
        pltpu.VMEM((CHUNK, D), jnp.float32),
        pltpu.VMEM((CHUNK, L), jnp.float32),
        pltpu.VMEM((CHUNK, L), jnp.float32),
        pltpu.SemaphoreType.DMA,
        pltpu.SemaphoreType.DMA,
        pltpu.SemaphoreType.DMA,
        pltpu.SemaphoreType.DMA,
        pltpu.SemaphoreType.DMA,
        pltpu.SemaphoreType.DMA,
    ],
)
def _edge_kernel(y_hbm, x_hbm, src_hbm, dst_hbm, agg_out, sum_out,
                 agg_sh, sum_sh, srcA, dstA, srcB, dstB, dsA, dsB, win_v,
                 yA, yB, xA, xB, eA, eB,
                 isemA, isemB, gsemA, gsemB, ssemA, ssemB):
    cid = lax.axis_index("c")
    sid = lax.axis_index("s")
    wid = sid * NC + cid
    tbase = (wid * NCHUNK + jnp.minimum(wid, 16)) * CHUNK

    zero16 = jnp.zeros((L,), jnp.float32)
    iota16 = lax.iota(jnp.int32, L)
    zero16i = jnp.zeros((L,), jnp.int32)
    row0 = sid * RPT

    @pl.loop(0, CHUNK)
    def _(i):
        for t in range(D // L):
            yA[i, pl.ds(L * t, L)] = zero16
        eA[i, :] = zero16
        eB[i, :] = zero16

    @pl.loop(0, RPT // CHUNK)
    def _(r):
        w0 = row0 + CHUNK * r
        for t in range(CHUNK // L):
            win_v[pl.ds(L * t, L)] = iota16 + (w0 + L * t)
        pltpu.sync_copy(yA, agg_sh.at[win_v])
        pltpu.sync_copy(eA, sum_sh.at[win_v])
    plsc.subcore_barrier()

    def compute(yr, xr, er):
        for g in range(GROUPS):
            rowids = iota16 + (g * L)

            def dot_body(k, acc):
                colk = jnp.full((L,), 0, jnp.int32) + k
                a = plsc.load_gather(yr, [rowids, colk])
                b = plsc.load_gather(xr, [rowids, colk])
                return acc + a * b

            acc = lax.fori_loop(0, D, dot_body, zero16, unroll=8)
            e16 = jnp.exp(acc)
            plsc.store_scatter(er, [rowids, zero16i], e16)
            for j in range(L):
                ej = g * L + j
                eb = plsc.load_gather(
                    er, [jnp.full((L,), ej, jnp.int32), zero16i])
                for t in range(D // L):
                    sl = pl.ds(L * t, L)
                    yr[ej, sl] = yr[ej, sl] * eb

    def idx_fetch(c, sv, dv, sem):
        d1 = pltpu.make_async_copy(src_hbm.at[pl.ds(tbase + c * CHUNK, CHUNK)],
                                   sv, sem)
        d2 = pltpu.make_async_copy(dst_hbm.at[pl.ds(tbase + c * CHUNK, CHUNK)],
                                   dv, sem)
        d1.start()
        d2.start()
        return d1, d2

    def idx_wait(sv, dv, sem):
        pltpu.make_async_copy(src_hbm.at[pl.ds(0, CHUNK)], sv, sem).wait()
        pltpu.make_async_copy(src_hbm.at[pl.ds(0, CHUNK)], dv, sem).wait()

    def rows_start(sv, dv, yr, xr, sem):
        d1 = pltpu.make_async_copy(y_hbm.at[sv], yr, sem)
        d2 = pltpu.make_async_copy(x_hbm.at[dv], xr, sem)
        d1.start()
        d2.start()

    def rows_wait(sv, dv, yr, xr, sem):
        pltpu.make_async_copy(y_hbm.at[sv], yr, sem).wait()
        pltpu.make_async_copy(x_hbm.at[dv], xr, sem).wait()

    def scat_start(yr, er, dsv, sem):
        d1 = pltpu.make_async_copy(yr, agg_sh.at[dsv], sem)
        d2 = pltpu.make_async_copy(er, sum_sh.at[dsv], sem)
        d1.start(add=True)
        d2.start(add=True)

    def scat_wait(yr, er, dsv, sem):
        pltpu.make_async_copy(yr, agg_sh.at[dsv], sem).wait()
        pltpu.make_async_copy(er, sum_sh.at[dsv], sem).wait()

    def keep_dst(dv, dsv):
        for t in range(CHUNK // L):
            sl = pl.ds(L * t, L)
            dsv[sl] = dv[sl]

    pltpu.sync_copy(src_hbm.at[pl.ds(tbase, CHUNK)], srcA)
    pltpu.sync_copy(dst_hbm.at[pl.ds(tbase, CHUNK)], dstA)
    pltpu.sync_copy(src_hbm.at[pl.ds(tbase + CHUNK, CHUNK)], srcB)
    pltpu.sync_copy(dst_hbm.at[pl.ds(tbase + CHUNK, CHUNK)], dstB)
    rows_start(srcA, dstA, yA, xA, gsemA)

    @pl.loop(0, NPAIR)
    def _(i):
        a = 2 * i
        rows_wait(srcA, dstA, yA, xA, gsemA)
        keep_dst(dstA, dsA)
        idx_fetch(a + 2, srcA, dstA, isemA)

        @pl.when(i > 0)
        def _():
            scat_wait(yB, eB, dsB, ssemB)
            idx_wait(srcB, dstB, isemB)

        rows_start(srcB, dstB, yB, xB, gsemB)
        compute(yA, xA, eA)
        scat_start(yA, eA, dsA, ssemA)

        rows_wait(srcB, dstB, yB, xB, gsemB)
        keep_dst(dstB, dsB)
        idx_fetch(a + 3, srcB, dstB, isemB)
        compute(yB, xB, eB)
        scat_start(yB, eB, dsB, ssemB)

        scat_wait(yA, eA, dsA, ssemA)
        idx_wait(srcA, dstA, isemA)
        rows_start(srcA, dstA, yA, xA, gsemA)

    rows_wait(srcA, dstA, yA, xA, gsemA)
    idx_wait(srcB, dstB, isemB)
    scat_wait(yB, eB, dsB, ssemB)

    @pl.when(wid < 16)
    def _():
        compute(yA, xA, eA)
        keep_dst(dstA, dsA)
        scat_start(yA, eA, dsA, ssemA)
        scat_wait(yA, eA, dsA, ssemA)

    plsc.subcore_barrier()

    @pl.loop(0, RPT // CHUNK)
    def _(r):
        w0 = row0 + CHUNK * r
        for t in range(CHUNK // L):
            win_v[pl.ds(L * t, L)] = iota16 + (w0 + L * t)
        pltpu.sync_copy(agg_sh.at[win_v], yA)
        pltpu.sync_copy(sum_sh.at[win_v], eA)
        pltpu.sync_copy(yA, agg_out.at[cid, pl.ds(w0, CHUNK)])
        pltpu.sync_copy(eA, sum_out.at[cid, pl.ds(w0, CHUNK)])


def _combine_body(pa_ref, ps_ref, o_ref):
    agg = pa_ref[0] + pa_ref[1]
    s = ps_ref[0, :, 0:1] + ps_ref[1, :, 0:1]
    agg = agg / (s + 1e-16)
    n2 = jnp.sum(agg * agg, axis=-1, keepdims=True)
    o_ref[...] = agg * (n2 / (1.0 + n2) / jnp.sqrt(n2 + 1e-9))


def _combine(pa, ps):
    return pl.pallas_call(
        _combine_body,
        out_shape=jax.ShapeDtypeStruct((NP, D), jnp.float32),
        grid=(5,),
        in_specs=[
            pl.BlockSpec((NC, NP // 5, D), lambda i: (0, i, 0)),
            pl.BlockSpec((NC, NP // 5, L), lambda i: (0, i, 0)),
        ],
        out_specs=pl.BlockSpec((NP // 5, D), lambda i: (i, 0)),
    )(pa, ps)


def kernel(x, edge_index, W):
    ei = edge_index.astype(jnp.int32)
    src = jnp.pad(ei[0], (0, EP - E))
    dst = jnp.pad(ei[1], (0, EP - E))
    y = _transform(x, W)
    pa, ps = _edge_kernel(y, x, src, dst)
    return _combine(pa, ps)[:N]


# speedup vs baseline: 5.4979x; 2.7362x over previous
import dataclasses
import functools

import jax
import jax.numpy as jnp
from jax import lax
from jax.experimental import pallas as pl
from jax.experimental.pallas import tpu as pltpu
from jax.experimental.pallas import tpu_sc as plsc

N = 10000
NP = 10240
E = 320000
EP = E + 1024
D = 128
L = 16
NC = 2
NS = 16
NW = NC * NS
CHUNK = 32
NCHUNK = 312
NPAIR = NCHUNK // 2
GROUPS = CHUNK // L
RPT = NP // NS


def _matmul_body(x_ref, w_ref, o_ref):
    o_ref[...] = jnp.dot(x_ref[...], w_ref[...],
                         preferred_element_type=jnp.float32)


def _transform(x, W):
    return pl.pallas_call(
        _matmul_body,
        out_shape=jax.ShapeDtypeStruct((N, D), jnp.float32),
        grid=(10,),
        in_specs=[
            pl.BlockSpec((N // 10, D), lambda i: (i, 0)),
            pl.BlockSpec((D, D), lambda i: (0, 0)),
        ],
        out_specs=pl.BlockSpec((N // 10, D), lambda i: (i, 0)),
    )(x, W)


_SC_PARAMS = pltpu.CompilerParams()
if "needs_layout_passes" in pltpu.CompilerParams.__dataclass_fields__:
    _SC_PARAMS = dataclasses.replace(_SC_PARAMS, needs_layout_passes=False)


@functools.partial(
    pl.kernel,
    compiler_params=_SC_PARAMS,
    out_type=(
        jax.ShapeDtypeStruct((NC, NP, D), jnp.float32),
        jax.ShapeDtypeStruct((NC, NP, L), jnp.float32),
    ),
    mesh=plsc.VectorSubcoreMesh(core_axis_name="c", subcore_axis_name="s",
                                num_cores=NC, num_subcores=NS),
    scratch_types=[
        pltpu.VMEM_SHARED((NP, D), jnp.float32),
        pltpu.VMEM_SHARED((NP, L), jnp.float32),
        pltpu.VMEM((CHUNK,), jnp.int32),
        pltpu.VMEM((CHUNK,), jnp.int32),
        pltpu.VMEM((CHUNK,), jnp.int32),
        pltpu.VMEM((CHUNK,), jnp.int32),
        pltpu.VMEM((CHUNK,), jnp.int32),
        pltpu.VMEM((CHUNK,), jnp.int32),
        pltpu.VMEM((CHUNK,), jnp.int32),
        pltpu.VMEM((CHUNK, D), jnp.float32),
        pltpu.VMEM((CHUNK, D), jnp.float32),
        pltpu.VMEM((CHUNK, D), jnp.float32),
        pltpu.VMEM((CHUNK, D), jnp.float32),
        pltpu.VMEM((CHUNK, L), jnp.float32),
        pltpu.VMEM((CHUNK, L), jnp.float32),
        pltpu.SemaphoreType.DMA,
        pltpu.SemaphoreType.DMA,
        pltpu.SemaphoreType.DMA,
        pltpu.SemaphoreType.DMA,
        pltpu.SemaphoreType.DMA,
        pltpu.SemaphoreType.DMA,
    ],
)
def _edge_kernel(y_hbm, x_hbm, src_hbm, dst_hbm, agg_out, sum_out,
                 agg_sh, sum_sh, srcA, dstA, srcB, dstB, dsA, dsB, win_v,
                 yA, yB, xA, xB, eA, eB,
                 isemA, isemB, gsemA, gsemB, ssemA, ssemB):
    cid = lax.axis_index("c")
    sid = lax.axis_index("s")
    wid = sid * NC + cid
    tbase = (wid * NCHUNK + jnp.minimum(wid, 16)) * CHUNK

    zero16 = jnp.zeros((L,), jnp.float32)
    iota16 = lax.iota(jnp.int32, L)
    zero16i = jnp.zeros((L,), jnp.int32)
    row0 = sid * RPT

    @pl.loop(0, CHUNK)
    def _(i):
        for t in range(D // L):
            yA[i, pl.ds(L * t, L)] = zero16
        eA[i, :] = zero16
        eB[i, :] = zero16

    @pl.loop(0, RPT // CHUNK)
    def _(r):
        w0 = row0 + CHUNK * r
        for t in range(CHUNK // L):
            win_v[pl.ds(L * t, L)] = iota16 + (w0 + L * t)
        pltpu.sync_copy(yA, agg_sh.at[win_v])
        pltpu.sync_copy(eA, sum_sh.at[win_v])
    plsc.subcore_barrier()

    def compute(yr, xr, er):
        @pl.loop(0, CHUNK, unroll=2)
        def _(j):
            ys = [yr[j, pl.ds(L * t, L)] for t in range(D // L)]
            xs = [xr[j, pl.ds(L * t, L)] for t in range(D // L)]
            acc = ys[0] * xs[0]
            for t in range(1, D // L):
                acc = acc + ys[t] * xs[t]
            s = lax.reduce_sum(acc, axes=(0,))
            eb = jnp.exp(jnp.full((L,), s, jnp.float32))
            er[j, :] = eb
            for t in range(D // L):
                yr[j, pl.ds(L * t, L)] = ys[t] * eb

    def idx_fetch(c, sv, dv, sem):
        d1 = pltpu.make_async_copy(src_hbm.at[pl.ds(tbase + c * CHUNK, CHUNK)],
                                   sv, sem)
        d2 = pltpu.make_async_copy(dst_hbm.at[pl.ds(tbase + c * CHUNK, CHUNK)],
                                   dv, sem)
        d1.start()
        d2.start()
        return d1, d2

    def idx_wait(sv, dv, sem):
        pltpu.make_async_copy(src_hbm.at[pl.ds(0, CHUNK)], sv, sem).wait()
        pltpu.make_async_copy(src_hbm.at[pl.ds(0, CHUNK)], dv, sem).wait()

    def rows_start(sv, dv, yr, xr, sem):
        d1 = pltpu.make_async_copy(y_hbm.at[sv], yr, sem)
        d2 = pltpu.make_async_copy(x_hbm.at[dv], xr, sem)
        d1.start()
        d2.start()

    def rows_wait(sv, dv, yr, xr, sem):
        pltpu.make_async_copy(y_hbm.at[sv], yr, sem).wait()
        pltpu.make_async_copy(x_hbm.at[dv], xr, sem).wait()

    def scat_start(yr, er, dsv, sem):
        d1 = pltpu.make_async_copy(yr, agg_sh.at[dsv], sem)
        d2 = pltpu.make_async_copy(er, sum_sh.at[dsv], sem)
        d1.start(add=True)
        d2.start(add=True)

    def scat_wait(yr, er, dsv, sem):
        pltpu.make_async_copy(yr, agg_sh.at[dsv], sem).wait()
        pltpu.make_async_copy(er, sum_sh.at[dsv], sem).wait()

    def keep_dst(dv, dsv):
        for t in range(CHUNK // L):
            sl = pl.ds(L * t, L)
            dsv[sl] = dv[sl]

    pltpu.sync_copy(src_hbm.at[pl.ds(tbase, CHUNK)], srcA)
    pltpu.sync_copy(dst_hbm.at[pl.ds(tbase, CHUNK)], dstA)
    pltpu.sync_copy(src_hbm.at[pl.ds(tbase + CHUNK, CHUNK)], srcB)
    pltpu.sync_copy(dst_hbm.at[pl.ds(tbase + CHUNK, CHUNK)], dstB)
    rows_start(srcA, dstA, yA, xA, gsemA)

    @pl.loop(0, NPAIR)
    def _(i):
        a = 2 * i
        rows_wait(srcA, dstA, yA, xA, gsemA)
        keep_dst(dstA, dsA)
        idx_fetch(a + 2, srcA, dstA, isemA)

        @pl.when(i > 0)
        def _():
            scat_wait(yB, eB, dsB, ssemB)
            idx_wait(srcB, dstB, isemB)

        rows_start(srcB, dstB, yB, xB, gsemB)
        compute(yA, xA, eA)
        scat_start(yA, eA, dsA, ssemA)

        rows_wait(srcB, dstB, yB, xB, gsemB)
        keep_dst(dstB, dsB)
        idx_fetch(a + 3, srcB, dstB, isemB)
        compute(yB, xB, eB)
        scat_start(yB, eB, dsB, ssemB)

        scat_wait(yA, eA, dsA, ssemA)
        idx_wait(srcA, dstA, isemA)
        rows_start(srcA, dstA, yA, xA, gsemA)

    rows_wait(srcA, dstA, yA, xA, gsemA)
    idx_wait(srcB, dstB, isemB)
    scat_wait(yB, eB, dsB, ssemB)

    @pl.when(wid < 16)
    def _():
        compute(yA, xA, eA)
        keep_dst(dstA, dsA)
        scat_start(yA, eA, dsA, ssemA)
        scat_wait(yA, eA, dsA, ssemA)

    plsc.subcore_barrier()

    @pl.loop(0, RPT // CHUNK)
    def _(r):
        w0 = row0 + CHUNK * r
        for t in range(CHUNK // L):
            win_v[pl.ds(L * t, L)] = iota16 + (w0 + L * t)
        pltpu.sync_copy(agg_sh.at[win_v], yA)
        pltpu.sync_copy(sum_sh.at[win_v], eA)
        pltpu.sync_copy(yA, agg_out.at[cid, pl.ds(w0, CHUNK)])
        pltpu.sync_copy(eA, sum_out.at[cid, pl.ds(w0, CHUNK)])


def _combine_body(pa_ref, ps_ref, o_ref):
    agg = pa_ref[0] + pa_ref[1]
    s = ps_ref[0, :, 0:1] + ps_ref[1, :, 0:1]
    agg = agg / (s + 1e-16)
    n2 = jnp.sum(agg * agg, axis=-1, keepdims=True)
    o_ref[...] = agg * (n2 / (1.0 + n2) / jnp.sqrt(n2 + 1e-9))


def _combine(pa, ps):
    return pl.pallas_call(
        _combine_body,
        out_shape=jax.ShapeDtypeStruct((NP, D), jnp.float32),
        grid=(5,),
        in_specs=[
            pl.BlockSpec((NC, NP // 5, D), lambda i: (0, i, 0)),
            pl.BlockSpec((NC, NP // 5, L), lambda i: (0, i, 0)),
        ],
        out_specs=pl.BlockSpec((NP // 5, D), lambda i: (i, 0)),
    )(pa, ps)


def kernel(x, edge_index, W):
    ei = edge_index.astype(jnp.int32)
    src = jnp.pad(ei[0], (0, EP - E))
    dst = jnp.pad(ei[1], (0, EP - E))
    y = _transform(x, W)
    pa, ps = _edge_kernel(y, x, src, dst)
    return _combine(pa, ps)[:N]


# speedup vs baseline: 5.5349x; 1.0067x over previous
import dataclasses
import functools

import jax
import jax.numpy as jnp
from jax import lax
from jax.experimental import pallas as pl
from jax.experimental.pallas import tpu as pltpu
from jax.experimental.pallas import tpu_sc as plsc

N = 10000
NP = 10240
E = 320000
EP = E + 1024
D = 128
L = 16
NC = 2
NS = 16
NW = NC * NS
CHUNK = 32
NCHUNK = 312
NPAIR = NCHUNK // 2
GROUPS = CHUNK // L
RPT = NP // NS


def _matmul_body(x_ref, w_ref, o_ref):
    o_ref[...] = jnp.dot(x_ref[...], w_ref[...],
                         preferred_element_type=jnp.float32)


def _transform(x, W):
    return pl.pallas_call(
        _matmul_body,
        out_shape=jax.ShapeDtypeStruct((N, D), jnp.float32),
        grid=(10,),
        in_specs=[
            pl.BlockSpec((N // 10, D), lambda i: (i, 0)),
            pl.BlockSpec((D, D), lambda i: (0, 0)),
        ],
        out_specs=pl.BlockSpec((N // 10, D), lambda i: (i, 0)),
    )(x, W)


_SC_PARAMS = pltpu.CompilerParams()
if "needs_layout_passes" in pltpu.CompilerParams.__dataclass_fields__:
    _SC_PARAMS = dataclasses.replace(_SC_PARAMS, needs_layout_passes=False)


@functools.partial(
    pl.kernel,
    compiler_params=_SC_PARAMS,
    out_type=(
        jax.ShapeDtypeStruct((NC, NP, D), jnp.float32),
        jax.ShapeDtypeStruct((NC, NP, L), jnp.float32),
    ),
    mesh=plsc.VectorSubcoreMesh(core_axis_name="c", subcore_axis_name="s",
                                num_cores=NC, num_subcores=NS),
    scratch_types=[
        pltpu.VMEM_SHARED((NP, D), jnp.float32),
        pltpu.VMEM_SHARED((NP, L), jnp.float32),
        pltpu.VMEM((CHUNK,), jnp.int32),
        pltpu.VMEM((CHUNK,), jnp.int32),
        pltpu.VMEM((CHUNK,), jnp.int32),
        pltpu.VMEM((CHUNK,), jnp.int32),
        pltpu.VMEM((CHUNK,), jnp.int32),
        pltpu.VMEM((CHUNK,), jnp.int32),
        pltpu.VMEM((CHUNK,), jnp.int32),
        pltpu.VMEM((CHUNK, D), jnp.float32),
        pltpu.VMEM((CHUNK, D), jnp.float32),
        pltpu.VMEM((CHUNK, D), jnp.float32),
        pltpu.VMEM((CHUNK, D), jnp.float32),
        pltpu.VMEM((CHUNK, L), jnp.float32),
        pltpu.VMEM((CHUNK, L), jnp.float32),
        pltpu.SemaphoreType.DMA,
        pltpu.SemaphoreType.DMA,
        pltpu.SemaphoreType.DMA,
        pltpu.SemaphoreType.DMA,
        pltpu.SemaphoreType.DMA,
        pltpu.SemaphoreType.DMA,
    ],
)
def _edge_kernel(y_hbm, x_hbm, src_hbm, dst_hbm, agg_out, sum_out,
                 agg_sh, sum_sh, srcA, dstA, srcB, dstB, dsA, dsB, win_v,
                 yA, yB, xA, xB, eA, eB,
                 isemA, isemB, gsemA, gsemB, ssemA, ssemB):
    cid = lax.axis_index("c")
    sid = lax.axis_index("s")
    wid = sid * NC + cid
    tbase = (wid * NCHUNK + jnp.minimum(wid, 16)) * CHUNK

    zero16 = jnp.zeros((L,), jnp.float32)
    iota16 = lax.iota(jnp.int32, L)
    zero16i = jnp.zeros((L,), jnp.int32)
    row0 = sid * RPT

    @pl.loop(0, CHUNK)
    def _(i):
        for t in range(D // L):
            yA[i, pl.ds(L * t, L)] = zero16
        eA[i, :] = zero16
        eB[i, :] = zero16

    @pl.loop(0, RPT // CHUNK)
    def _(r):
        w0 = row0 + CHUNK * r
        for t in range(CHUNK // L):
            win_v[pl.ds(L * t, L)] = iota16 + (w0 + L * t)
        pltpu.sync_copy(yA, agg_sh.at[win_v])
        pltpu.sync_copy(eA, sum_sh.at[win_v])
    plsc.subcore_barrier()

    def compute(yr, xr, er):
        @pl.loop(0, CHUNK, unroll=4)
        def _(j):
            ys = [yr[j, pl.ds(L * t, L)] for t in range(D // L)]
            xs = [xr[j, pl.ds(L * t, L)] for t in range(D // L)]
            acc = ys[0] * xs[0]
            for t in range(1, D // L):
                acc = acc + ys[t] * xs[t]
            s = lax.reduce_sum(acc, axes=(0,))
            eb = jnp.exp(jnp.full((L,), s, jnp.float32))
            er[j, :] = eb
            for t in range(D // L):
                yr[j, pl.ds(L * t, L)] = ys[t] * eb

    def idx_fetch(c, sv, dv, sem):
        d1 = pltpu.make_async_copy(src_hbm.at[pl.ds(tbase + c * CHUNK, CHUNK)],
                                   sv, sem)
        d2 = pltpu.make_async_copy(dst_hbm.at[pl.ds(tbase + c * CHUNK, CHUNK)],
                                   dv, sem)
        d1.start()
        d2.start()
        return d1, d2

    def idx_wait(sv, dv, sem):
        pltpu.make_async_copy(src_hbm.at[pl.ds(0, CHUNK)], sv, sem).wait()
        pltpu.make_async_copy(src_hbm.at[pl.ds(0, CHUNK)], dv, sem).wait()

    def rows_start(sv, dv, yr, xr, sem):
        d1 = pltpu.make_async_copy(y_hbm.at[sv], yr, sem)
        d2 = pltpu.make_async_copy(x_hbm.at[dv], xr, sem)
        d1.start()
        d2.start()

    def rows_wait(sv, dv, yr, xr, sem):
        pltpu.make_async_copy(y_hbm.at[sv], yr, sem).wait()
        pltpu.make_async_copy(x_hbm.at[dv], xr, sem).wait()

    def scat_start(yr, er, dsv, sem):
        d1 = pltpu.make_async_copy(yr, agg_sh.at[dsv], sem)
        d2 = pltpu.make_async_copy(er, sum_sh.at[dsv], sem)
        d1.start(add=True)
        d2.start(add=True)

    def scat_wait(yr, er, dsv, sem):
        pltpu.make_async_copy(yr, agg_sh.at[dsv], sem).wait()
        pltpu.make_async_copy(er, sum_sh.at[dsv], sem).wait()

    def keep_dst(dv, dsv):
        for t in range(CHUNK // L):
            sl = pl.ds(L * t, L)
            dsv[sl] = dv[sl]

    pltpu.sync_copy(src_hbm.at[pl.ds(tbase, CHUNK)], srcA)
    pltpu.sync_copy(dst_hbm.at[pl.ds(tbase, CHUNK)], dstA)
    pltpu.sync_copy(src_hbm.at[pl.ds(tbase + CHUNK, CHUNK)], srcB)
    pltpu.sync_copy(dst_hbm.at[pl.ds(tbase + CHUNK, CHUNK)], dstB)
    rows_start(srcA, dstA, yA, xA, gsemA)

    @pl.loop(0, NPAIR)
    def _(i):
        a = 2 * i
        rows_wait(srcA, dstA, yA, xA, gsemA)
        keep_dst(dstA, dsA)
        idx_fetch(a + 2, srcA, dstA, isemA)

        @pl.when(i > 0)
        def _():
            scat_wait(yB, eB, dsB, ssemB)
            idx_wait(srcB, dstB, isemB)

        rows_start(srcB, dstB, yB, xB, gsemB)
        compute(yA, xA, eA)
        scat_start(yA, eA, dsA, ssemA)

        rows_wait(srcB, dstB, yB, xB, gsemB)
        keep_dst(dstB, dsB)
        idx_fetch(a + 3, srcB, dstB, isemB)
        compute(yB, xB, eB)
        scat_start(yB, eB, dsB, ssemB)

        scat_wait(yA, eA, dsA, ssemA)
        idx_wait(srcA, dstA, isemA)
        rows_start(srcA, dstA, yA, xA, gsemA)

    rows_wait(srcA, dstA, yA, xA, gsemA)
    idx_wait(srcB, dstB, isemB)
    scat_wait(yB, eB, dsB, ssemB)

    @pl.when(wid < 16)
    def _():
        compute(yA, xA, eA)
        keep_dst(dstA, dsA)
        scat_start(yA, eA, dsA, ssemA)
        scat_wait(yA, eA, dsA, ssemA)

    plsc.subcore_barrier()

    @pl.loop(0, RPT // CHUNK)
    def _(r):
        w0 = row0 + CHUNK * r
        for t in range(CHUNK // L):
            win_v[pl.ds(L * t, L)] = iota16 + (w0 + L * t)
        pltpu.sync_copy(agg_sh.at[win_v], yA)
        pltpu.sync_copy(sum_sh.at[win_v], eA)
        pltpu.sync_copy(yA, agg_out.at[cid, pl.ds(w0, CHUNK)])
        pltpu.sync_copy(eA, sum_out.at[cid, pl.ds(w0, CHUNK)])


def _combine_body(pa_ref, ps_ref, o_ref):
    agg = pa_ref[0] + pa_ref[1]
    s = ps_ref[0, :, 0:1] + ps_ref[1, :, 0:1]
    agg = agg / (s + 1e-16)
    n2 = jnp.sum(agg * agg, axis=-1, keepdims=True)
    o_ref[...] = agg * (n2 / (1.0 + n2) / jnp.sqrt(n2 + 1e-9))


def _combine(pa, ps):
    return pl.pallas_call(
        _combine_body,
        out_shape=jax.ShapeDtypeStruct((NP, D), jnp.float32),
        grid=(5,),
        in_specs=[
            pl.BlockSpec((NC, NP // 5, D), lambda i: (0, i, 0)),
            pl.BlockSpec((NC, NP // 5, L), lambda i: (0, i, 0)),
        ],
        out_specs=pl.BlockSpec((NP // 5, D), lambda i: (i, 0)),
    )(pa, ps)


def kernel(x, edge_index, W):
    ei = edge_index.astype(jnp.int32)
    src = jnp.pad(ei[0], (0, EP - E))
    dst = jnp.pad(ei[1], (0, EP - E))
    y = _transform(x, W)
    pa, ps = _edge_kernel(y, x, src, dst)
    return _combine(pa, ps)[:N]


# speedup vs baseline: 6.5313x; 1.1800x over previous
import dataclasses
import functools

import jax
import jax.numpy as jnp
from jax import lax
from jax.experimental import pallas as pl
from jax.experimental.pallas import tpu as pltpu
from jax.experimental.pallas import tpu_sc as plsc

N = 10000
NP = 10240
E = 320000
EP = E + 1024
D = 128
L = 16
NC = 2
NS = 16
NW = NC * NS
CHUNK = 32
NCHUNK = 312
NPAIR = NCHUNK // 2
GROUPS = CHUNK // L
RPT = NP // NS


def _matmul_body(x_ref, w_ref, o_ref):
    o_ref[...] = jnp.dot(x_ref[...], w_ref[...],
                         preferred_element_type=jnp.float32)


def _transform(x, W):
    return pl.pallas_call(
        _matmul_body,
        out_shape=jax.ShapeDtypeStruct((N, D), jnp.float32),
        grid=(10,),
        in_specs=[
            pl.BlockSpec((N // 10, D), lambda i: (i, 0)),
            pl.BlockSpec((D, D), lambda i: (0, 0)),
        ],
        out_specs=pl.BlockSpec((N // 10, D), lambda i: (i, 0)),
    )(x, W)


_SC_PARAMS = pltpu.CompilerParams()
if "needs_layout_passes" in pltpu.CompilerParams.__dataclass_fields__:
    _SC_PARAMS = dataclasses.replace(_SC_PARAMS, needs_layout_passes=False)


@functools.partial(
    pl.kernel,
    compiler_params=_SC_PARAMS,
    out_type=(
        jax.ShapeDtypeStruct((NC, NP, D), jnp.float32),
        jax.ShapeDtypeStruct((NC, NP, L), jnp.float32),
    ),
    mesh=plsc.VectorSubcoreMesh(core_axis_name="c", subcore_axis_name="s",
                                num_cores=NC, num_subcores=NS),
    scratch_types=[
        pltpu.VMEM_SHARED((NP, D), jnp.float32),
        pltpu.VMEM_SHARED((NP, L), jnp.float32),
        pltpu.VMEM((CHUNK,), jnp.int32),
        pltpu.VMEM((CHUNK,), jnp.int32),
        pltpu.VMEM((CHUNK,), jnp.int32),
        pltpu.VMEM((CHUNK,), jnp.int32),
        pltpu.VMEM((CHUNK,), jnp.int32),
        pltpu.VMEM((CHUNK,), jnp.int32),
        pltpu.VMEM((CHUNK,), jnp.int32),
        pltpu.VMEM((CHUNK, D), jnp.float32),
        pltpu.VMEM((CHUNK, D), jnp.float32),
        pltpu.VMEM((CHUNK, D), jnp.float32),
        pltpu.VMEM((CHUNK, D), jnp.float32),
        pltpu.VMEM((CHUNK, L), jnp.float32),
        pltpu.VMEM((CHUNK, L), jnp.float32),
        pltpu.SemaphoreType.DMA,
        pltpu.SemaphoreType.DMA,
        pltpu.SemaphoreType.DMA,
        pltpu.SemaphoreType.DMA,
        pltpu.SemaphoreType.DMA,
        pltpu.SemaphoreType.DMA,
    ],
)
def _edge_kernel(y_hbm, x_hbm, src_hbm, dst_hbm, agg_out, sum_out,
                 agg_sh, sum_sh, srcA, dstA, srcB, dstB, dsA, dsB, win_v,
                 yA, yB, xA, xB, eA, eB,
                 isemA, isemB, gsemA, gsemB, ssemA, ssemB):
    cid = lax.axis_index("c")
    sid = lax.axis_index("s")
    wid = sid * NC + cid
    tbase = (wid * NCHUNK + jnp.minimum(wid, 16)) * CHUNK

    zero16 = jnp.zeros((L,), jnp.float32)
    iota16 = lax.iota(jnp.int32, L)
    zero16i = jnp.zeros((L,), jnp.int32)
    row0 = sid * RPT

    @pl.loop(0, CHUNK)
    def _(i):
        for t in range(D // L):
            yA[i, pl.ds(L * t, L)] = zero16
        eA[i, :] = zero16
        eB[i, :] = zero16

    @pl.loop(0, RPT // CHUNK)
    def _(r):
        w0 = row0 + CHUNK * r
        for t in range(CHUNK // L):
            win_v[pl.ds(L * t, L)] = iota16 + (w0 + L * t)
        pltpu.sync_copy(yA, agg_sh.at[win_v])
        pltpu.sync_copy(eA, sum_sh.at[win_v])
    plsc.subcore_barrier()

    def compute(yr, xr, er):
        @pl.loop(0, CHUNK, unroll=4)
        def _(j):
            ys = [yr[j, pl.ds(L * t, L)] for t in range(D // L)]
            xs = [xr[j, pl.ds(L * t, L)] for t in range(D // L)]
            acc = ys[0] * xs[0]
            for t in range(1, D // L):
                acc = acc + ys[t] * xs[t]
            s = lax.reduce_sum(acc, axes=(0,))
            eb = jnp.exp(jnp.full((L,), s, jnp.float32))
            er[j, :] = eb
            for t in range(D // L):
                yr[j, pl.ds(L * t, L)] = ys[t] * eb

    def idx_fetch(c, sv, dv, sem):
        d1 = pltpu.make_async_copy(src_hbm.at[pl.ds(tbase + c * CHUNK, CHUNK)],
                                   sv, sem)
        d2 = pltpu.make_async_copy(dst_hbm.at[pl.ds(tbase + c * CHUNK, CHUNK)],
                                   dv, sem)
        d1.start()
        d2.start()
        return d1, d2

    def idx_wait(sv, dv, sem):
        pltpu.make_async_copy(src_hbm.at[pl.ds(0, CHUNK)], sv, sem).wait()
        pltpu.make_async_copy(src_hbm.at[pl.ds(0, CHUNK)], dv, sem).wait()

    def rows_start(sv, dv, yr, xr, sem):
        d1 = pltpu.make_async_copy(y_hbm.at[sv], yr, sem)
        d2 = pltpu.make_async_copy(x_hbm.at[dv], xr, sem)
        d1.start()
        d2.start()

    def rows_wait(sv, dv, yr, xr, sem):
        pltpu.make_async_copy(y_hbm.at[sv], yr, sem).wait()
        pltpu.make_async_copy(x_hbm.at[dv], xr, sem).wait()

    def scat_start(yr, er, dsv, sem):
        d1 = pltpu.make_async_copy(yr, agg_sh.at[dsv], sem)
        d2 = pltpu.make_async_copy(er, sum_sh.at[dsv], sem)
        d1.start(add=True)
        d2.start(add=True)

    def scat_wait(yr, er, dsv, sem):
        pltpu.make_async_copy(yr, agg_sh.at[dsv], sem).wait()
        pltpu.make_async_copy(er, sum_sh.at[dsv], sem).wait()

    def keep_dst(dv, dsv):
        for t in range(CHUNK // L):
            sl = pl.ds(L * t, L)
            dsv[sl] = dv[sl]

    pltpu.sync_copy(src_hbm.at[pl.ds(tbase, CHUNK)], srcA)
    pltpu.sync_copy(dst_hbm.at[pl.ds(tbase, CHUNK)], dstA)
    pltpu.sync_copy(src_hbm.at[pl.ds(tbase + CHUNK, CHUNK)], srcB)
    pltpu.sync_copy(dst_hbm.at[pl.ds(tbase + CHUNK, CHUNK)], dstB)
    rows_start(srcA, dstA, yA, xA, gsemA)

    @pl.loop(0, NPAIR)
    def _(i):
        a = 2 * i
        rows_wait(srcA, dstA, yA, xA, gsemA)
        keep_dst(dstA, dsA)
        idx_fetch(a + 2, srcA, dstA, isemA)

        @pl.when(i > 0)
        def _():
            scat_wait(yB, eB, dsB, ssemB)
            idx_wait(srcB, dstB, isemB)

        rows_start(srcB, dstB, yB, xB, gsemB)
        compute(yA, xA, eA)
        scat_start(yA, eA, dsA, ssemA)
        scat_wait(yA, eA, dsA, ssemA)
        idx_wait(srcA, dstA, isemA)
        rows_start(srcA, dstA, yA, xA, gsemA)

        rows_wait(srcB, dstB, yB, xB, gsemB)
        keep_dst(dstB, dsB)
        idx_fetch(a + 3, srcB, dstB, isemB)
        compute(yB, xB, eB)
        scat_start(yB, eB, dsB, ssemB)

    rows_wait(srcA, dstA, yA, xA, gsemA)
    idx_wait(srcB, dstB, isemB)
    scat_wait(yB, eB, dsB, ssemB)

    @pl.when(wid < 16)
    def _():
        compute(yA, xA, eA)
        keep_dst(dstA, dsA)
        scat_start(yA, eA, dsA, ssemA)
        scat_wait(yA, eA, dsA, ssemA)

    plsc.subcore_barrier()

    @pl.loop(0, RPT // CHUNK)
    def _(r):
        w0 = row0 + CHUNK * r
        for t in range(CHUNK // L):
            win_v[pl.ds(L * t, L)] = iota16 + (w0 + L * t)
        pltpu.sync_copy(agg_sh.at[win_v], yA)
        pltpu.sync_copy(sum_sh.at[win_v], eA)
        pltpu.sync_copy(yA, agg_out.at[cid, pl.ds(w0, CHUNK)])
        pltpu.sync_copy(eA, sum_out.at[cid, pl.ds(w0, CHUNK)])


def _combine_body(pa_ref, ps_ref, o_ref):
    agg = pa_ref[0] + pa_ref[1]
    s = ps_ref[0, :, 0:1] + ps_ref[1, :, 0:1]
    agg = agg / (s + 1e-16)
    n2 = jnp.sum(agg * agg, axis=-1, keepdims=True)
    o_ref[...] = agg * (n2 / (1.0 + n2) / jnp.sqrt(n2 + 1e-9))


def _combine(pa, ps):
    return pl.pallas_call(
        _combine_body,
        out_shape=jax.ShapeDtypeStruct((NP, D), jnp.float32),
        grid=(5,),
        in_specs=[
            pl.BlockSpec((NC, NP // 5, D), lambda i: (0, i, 0)),
            pl.BlockSpec((NC, NP // 5, L), lambda i: (0, i, 0)),
        ],
        out_specs=pl.BlockSpec((NP // 5, D), lambda i: (i, 0)),
    )(pa, ps)


def kernel(x, edge_index, W):
    ei = edge_index.astype(jnp.int32)
    src = jnp.pad(ei[0], (0, EP - E))
    dst = jnp.pad(ei[1], (0, EP - E))
    y = _transform(x, W)
    pa, ps = _edge_kernel(y, x, src, dst)
    return _combine(pa, ps)[:N]


# speedup vs baseline: 9.0411x; 1.3843x over previous
import dataclasses
import functools

import jax
import jax.numpy as jnp
from jax import lax
from jax.experimental import pallas as pl
from jax.experimental.pallas import tpu as pltpu
from jax.experimental.pallas import tpu_sc as plsc

N = 10000
NP = 10240
E = 320000
EP = E + 1024
D = 128
L = 16
NC = 2
NS = 16
NW = NC * NS
CHUNK = 32
NCHUNK = 312
NPAIR = NCHUNK // 2
GROUPS = CHUNK // L
RPT = NP // NS


def _matmul_body(x_ref, w_ref, o_ref):
    o_ref[...] = jnp.dot(x_ref[...], w_ref[...],
                         preferred_element_type=jnp.float32)


def _transform(x, W):
    return pl.pallas_call(
        _matmul_body,
        out_shape=jax.ShapeDtypeStruct((N, D), jnp.float32),
        grid=(10,),
        in_specs=[
            pl.BlockSpec((N // 10, D), lambda i: (i, 0)),
            pl.BlockSpec((D, D), lambda i: (0, 0)),
        ],
        out_specs=pl.BlockSpec((N // 10, D), lambda i: (i, 0)),
    )(x, W)


_SC_PARAMS = pltpu.CompilerParams()
if "needs_layout_passes" in pltpu.CompilerParams.__dataclass_fields__:
    _SC_PARAMS = dataclasses.replace(_SC_PARAMS, needs_layout_passes=False)


@functools.partial(
    pl.kernel,
    compiler_params=_SC_PARAMS,
    out_type=(
        jax.ShapeDtypeStruct((NC, NP, D), jnp.float32),
        jax.ShapeDtypeStruct((NC, NP, L), jnp.float32),
    ),
    mesh=plsc.VectorSubcoreMesh(core_axis_name="c", subcore_axis_name="s",
                                num_cores=NC, num_subcores=NS),
    scratch_types=[
        pltpu.VMEM_SHARED((NP, D), jnp.float32),
        pltpu.VMEM_SHARED((NP, L), jnp.float32),
        pltpu.VMEM((CHUNK,), jnp.int32),
        pltpu.VMEM((CHUNK,), jnp.int32),
        pltpu.VMEM((CHUNK,), jnp.int32),
        pltpu.VMEM((CHUNK,), jnp.int32),
        pltpu.VMEM((CHUNK,), jnp.int32),
        pltpu.VMEM((CHUNK,), jnp.int32),
        pltpu.VMEM((CHUNK,), jnp.int32),
        pltpu.VMEM((CHUNK, D), jnp.float32),
        pltpu.VMEM((CHUNK, D), jnp.float32),
        pltpu.VMEM((CHUNK, D), jnp.float32),
        pltpu.VMEM((CHUNK, D), jnp.float32),
        pltpu.VMEM((CHUNK, L), jnp.float32),
        pltpu.VMEM((CHUNK, L), jnp.float32),
        pltpu.SemaphoreType.DMA,
        pltpu.SemaphoreType.DMA,
        pltpu.SemaphoreType.DMA,
        pltpu.SemaphoreType.DMA,
        pltpu.SemaphoreType.DMA,
        pltpu.SemaphoreType.DMA,
    ],
)
def _edge_kernel(y_hbm, x_hbm, src_hbm, dst_hbm, agg_out, sum_out,
                 agg_sh, sum_sh, srcA, dstA, srcB, dstB, dsA, dsB, win_v,
                 yA, yB, xA, xB, eA, eB,
                 isemA, isemB, gsemA, gsemB, ssemA, ssemB):
    cid = lax.axis_index("c")
    sid = lax.axis_index("s")
    wid = sid * NC + cid
    tbase = (wid * NCHUNK + jnp.minimum(wid, 16)) * CHUNK

    zero16 = jnp.zeros((L,), jnp.float32)
    iota16 = lax.iota(jnp.int32, L)
    zero16i = jnp.zeros((L,), jnp.int32)
    row0 = sid * RPT

    @pl.loop(0, CHUNK)
    def _(i):
        for t in range(D // L):
            yA[i, pl.ds(L * t, L)] = zero16
        eA[i, :] = zero16
        eB[i, :] = zero16

    @pl.loop(0, RPT // CHUNK)
    def _(r):
        w0 = row0 + CHUNK * r
        for t in range(CHUNK // L):
            win_v[pl.ds(L * t, L)] = iota16 + (w0 + L * t)
        pltpu.sync_copy(yA, agg_sh.at[win_v])
        pltpu.sync_copy(eA, sum_sh.at[win_v])
    plsc.subcore_barrier()

    def compute(yr, xr, er):
        @pl.loop(0, CHUNK, unroll=8)
        def _(j):
            ys = [yr[j, pl.ds(L * t, L)] for t in range(D // L)]
            xs = [xr[j, pl.ds(L * t, L)] for t in range(D // L)]
            acc = ys[0] * xs[0]
            for t in range(1, D // L):
                acc = acc + ys[t] * xs[t]
            s = lax.reduce_sum(acc, axes=(0,))
            eb = jnp.exp(jnp.full((L,), s, jnp.float32))
            er[j, :] = eb
            for t in range(D // L):
                yr[j, pl.ds(L * t, L)] = ys[t] * eb

    def idx_fetch(c, sv, dv, sem):
        d1 = pltpu.make_async_copy(src_hbm.at[pl.ds(tbase + c * CHUNK, CHUNK)],
                                   sv, sem)
        d2 = pltpu.make_async_copy(dst_hbm.at[pl.ds(tbase + c * CHUNK, CHUNK)],
                                   dv, sem)
        d1.start()
        d2.start()
        return d1, d2

    def idx_wait(sv, dv, sem):
        pltpu.make_async_copy(src_hbm.at[pl.ds(0, CHUNK)], sv, sem).wait()
        pltpu.make_async_copy(src_hbm.at[pl.ds(0, CHUNK)], dv, sem).wait()

    def rows_start(sv, dv, yr, xr, sem):
        d1 = pltpu.make_async_copy(y_hbm.at[sv], yr, sem)
        d2 = pltpu.make_async_copy(x_hbm.at[dv], xr, sem)
        d1.start()
        d2.start()

    def rows_wait(sv, dv, yr, xr, sem):
        pltpu.make_async_copy(y_hbm.at[sv], yr, sem).wait()
        pltpu.make_async_copy(x_hbm.at[dv], xr, sem).wait()

    def scat_start(yr, er, dsv, sem):
        d1 = pltpu.make_async_copy(yr, agg_sh.at[dsv], sem)
        d2 = pltpu.make_async_copy(er, sum_sh.at[dsv], sem)
        d1.start(add=True)
        d2.start(add=True)

    def scat_wait(yr, er, dsv, sem):
        pltpu.make_async_copy(yr, agg_sh.at[dsv], sem).wait()
        pltpu.make_async_copy(er, sum_sh.at[dsv], sem).wait()

    def keep_dst(dv, dsv):
        for t in range(CHUNK // L):
            sl = pl.ds(L * t, L)
            dsv[sl] = dv[sl]

    pltpu.sync_copy(src_hbm.at[pl.ds(tbase, CHUNK)], srcA)
    pltpu.sync_copy(dst_hbm.at[pl.ds(tbase, CHUNK)], dstA)
    pltpu.sync_copy(src_hbm.at[pl.ds(tbase + CHUNK, CHUNK)], srcB)
    pltpu.sync_copy(dst_hbm.at[pl.ds(tbase + CHUNK, CHUNK)], dstB)
    rows_start(srcA, dstA, yA, xA, gsemA)

    @pl.loop(0, NPAIR)
    def _(i):
        a = 2 * i
        rows_wait(srcA, dstA, yA, xA, gsemA)
        keep_dst(dstA, dsA)
        idx_fetch(a + 2, srcA, dstA, isemA)

        @pl.when(i > 0)
        def _():
            scat_wait(yB, eB, dsB, ssemB)
            idx_wait(srcB, dstB, isemB)

        rows_start(srcB, dstB, yB, xB, gsemB)
        compute(yA, xA, eA)
        scat_start(yA, eA, dsA, ssemA)
        scat_wait(yA, eA, dsA, ssemA)
        idx_wait(srcA, dstA, isemA)
        rows_start(srcA, dstA, yA, xA, gsemA)

        rows_wait(srcB, dstB, yB, xB, gsemB)
        keep_dst(dstB, dsB)
        idx_fetch(a + 3, srcB, dstB, isemB)
        compute(yB, xB, eB)
        scat_start(yB, eB, dsB, ssemB)

    rows_wait(srcA, dstA, yA, xA, gsemA)
    idx_wait(srcB, dstB, isemB)
    scat_wait(yB, eB, dsB, ssemB)

    @pl.when(wid < 16)
    def _():
        compute(yA, xA, eA)
        keep_dst(dstA, dsA)
        scat_start(yA, eA, dsA, ssemA)
        scat_wait(yA, eA, dsA, ssemA)

    plsc.subcore_barrier()

    @pl.loop(0, RPT // CHUNK)
    def _(r):
        w0 = row0 + CHUNK * r
        for t in range(CHUNK // L):
            win_v[pl.ds(L * t, L)] = iota16 + (w0 + L * t)
        pltpu.sync_copy(agg_sh.at[win_v], yA)
        pltpu.sync_copy(sum_sh.at[win_v], eA)
        pltpu.sync_copy(yA, agg_out.at[cid, pl.ds(w0, CHUNK)])
        pltpu.sync_copy(eA, sum_out.at[cid, pl.ds(w0, CHUNK)])


def _combine_body(pa_ref, ps_ref, o_ref):
    agg = pa_ref[0] + pa_ref[1]
    s = ps_ref[0, :, 0:1] + ps_ref[1, :, 0:1]
    agg = agg / (s + 1e-16)
    n2 = jnp.sum(agg * agg, axis=-1, keepdims=True)
    o_ref[...] = agg * (n2 / (1.0 + n2) / jnp.sqrt(n2 + 1e-9))


def _combine(pa, ps):
    return pl.pallas_call(
        _combine_body,
        out_shape=jax.ShapeDtypeStruct((NP, D), jnp.float32),
        grid=(5,),
        in_specs=[
            pl.BlockSpec((NC, NP // 5, D), lambda i: (0, i, 0)),
            pl.BlockSpec((NC, NP // 5, L), lambda i: (0, i, 0)),
        ],
        out_specs=pl.BlockSpec((NP // 5, D), lambda i: (i, 0)),
    )(pa, ps)


def kernel(x, edge_index, W):
    ei = edge_index.astype(jnp.int32)
    src = jnp.pad(ei[0], (0, EP - E))
    dst = jnp.pad(ei[1], (0, EP - E))
    y = _transform(x, W)
    pa, ps = _edge_kernel(y, x, src, dst)
    return _combine(pa, ps)[:N]
